# baseline (speedup 1.0000x reference)
import functools

import jax
import jax.numpy as jnp
from jax import lax
from jax.experimental import pallas as pl
from jax.experimental.pallas import tpu as pltpu
from jax.experimental.pallas import tpu_sc as plsc

F = 26
VOCAB = 100000
EMB = 32
B = 1024
T = 200

BT = B * T
NC = 2
NS = 16
NW = NC * NS
B_W = B // NW
Q = 40
TCH = T // Q
NCHUNK = B_W * TCH

_mesh = plsc.VectorSubcoreMesh(
    core_axis_name="c", subcore_axis_name="s", num_cores=NC, num_subcores=NS
)


@functools.partial(
    pl.kernel,
    mesh=_mesh,
    out_type=jax.ShapeDtypeStruct((B, T, F * EMB), jnp.float32),
    compiler_params=pltpu.CompilerParams(use_tc_tiling_on_sc=False),
    scratch_types=[
        pltpu.VMEM((F, Q), jnp.int32),
        pltpu.VMEM((F * Q, EMB), jnp.float32),
        pltpu.SemaphoreType.DMA,
        pltpu.SemaphoreType.DMA,
    ],
)
def _gather_kernel(tables_hbm, gidx_hbm, out_hbm, idx_v, stages, semg, semw):
    wid = lax.axis_index("s") * NC + lax.axis_index("c")

    def chunk_body(g, carry):
        b = wid * B_W + g // TCH
        t0 = (g % TCH) * Q
        bt0 = b * T + t0
        pltpu.sync_copy(gidx_hbm.at[:, pl.ds(bt0, Q)], idx_v)

        def fire_gather(f, c):
            pltpu.async_copy(
                tables_hbm.at[f].at[idx_v.at[f]],
                stages.at[pl.ds(f * Q, Q)],
                semg,
            )
            return c

        lax.fori_loop(0, F, fire_gather, 0)
        pltpu.make_async_copy(
            tables_hbm.at[0].at[pl.ds(0, F * Q)], stages, semg
        ).wait()

        def fire_write(f, c):
            pltpu.async_copy(
                stages.at[pl.ds(f * Q, Q)],
                out_hbm.at[b, pl.ds(t0, Q), pl.ds(f * EMB, EMB)],
                semw,
            )
            return c

        lax.fori_loop(0, F, fire_write, 0)
        pltpu.make_async_copy(stages, out_hbm.at[b, pl.ds(t0, Q)], semw).wait()
        return carry

    lax.fori_loop(0, NCHUNK, chunk_body, 0)


def kernel(tables, indices, seq_lens):
    gidx_t = indices.transpose(2, 0, 1).reshape(F, BT)
    return _gather_kernel(tables, gidx_t)
